# baseline (speedup 1.0000x reference)
import functools

import jax
import jax.numpy as jnp
from jax import lax
from jax.experimental import pallas as pl
from jax.experimental.pallas import tpu as pltpu
from jax.experimental.pallas import tpu_sc as plsc

_L = 16
_CHUNK = 128
_D = 8


@functools.lru_cache(maxsize=None)
def _sc_gather_prod(total, rank, table_rows):
    info = plsc.get_sparse_core_info()
    num_workers = info.num_cores * info.num_subcores
    per_w = total // num_workers
    assert per_w % _CHUNK == 0
    n_chunks = per_w // _CHUNK

    mesh = plsc.VectorSubcoreMesh(core_axis_name="c", subcore_axis_name="s")

    @functools.partial(
        pl.kernel,
        mesh=mesh,
        compiler_params=pltpu.CompilerParams(use_tc_tiling_on_sc=False),
        out_type=jax.ShapeDtypeStruct((total, _L), jnp.float32),
        scratch_types=[
            pltpu.VMEM((per_w,), jnp.float32),
            pltpu.VMEM((per_w,), jnp.float32),
            pltpu.VMEM((per_w,), jnp.float32),
            pltpu.VMEM((_CHUNK,), jnp.int32),
            pltpu.VMEM((_CHUNK,), jnp.int32),
            pltpu.VMEM((_CHUNK,), jnp.int32),
            pltpu.VMEM((_CHUNK, rank), jnp.float32),
            pltpu.VMEM((_CHUNK, rank), jnp.float32),
            pltpu.VMEM((_CHUNK, rank), jnp.float32),
            pltpu.VMEM((_CHUNK, _L), jnp.float32),
            pltpu.SemaphoreType.DMA,
        ],
    )
    def sc_fn(xf, yf, tf, At, Bt, Ct, p_out,
              fx, fy, ft, ix, iy, it, rA, rB, rC, pbuf, sem):
        wid = lax.axis_index("s") * info.num_cores + lax.axis_index("c")
        base = wid * per_w
        pltpu.sync_copy(xf.at[pl.ds(base, per_w)], fx)
        pltpu.sync_copy(yf.at[pl.ds(base, per_w)], fy)
        pltpu.sync_copy(tf.at[pl.ds(base, per_w)], ft)

        xscale = jnp.float32(table_rows - 1)
        yscale = jnp.float32(table_rows)
        hi = table_rows - 1
        lo = pl.ds(0, _L)
        hi_sl = pl.ds(_L, _L)

        def chunk_body(c, carry):
            coff = c * _CHUNK
            for gi in range(_CHUNK // _L):
                src = pl.ds(coff + gi * _L, _L)
                dst = pl.ds(gi * _L, _L)
                ix[dst] = jnp.clip((fx[src] * xscale).astype(jnp.int32), 0, hi)
                iy[dst] = jnp.clip((fy[src] * yscale - 1.0).astype(jnp.int32), 0, hi)
                it[dst] = jnp.clip((ft[src] * yscale - 1.0).astype(jnp.int32), 0, hi)
            ca = pltpu.async_copy(At.at[ix], rA, sem)
            cb = pltpu.async_copy(Bt.at[iy], rB, sem)
            cc = pltpu.async_copy(Ct.at[it], rC, sem)
            ca.wait()
            cb.wait()
            cc.wait()
            pending = {}
            for j in range(_CHUNK + _D):
                if j < _CHUNK:
                    pending[j] = (
                        rA[j, lo] * rB[j, lo] * rC[j, lo]
                        + rA[j, hi_sl] * rB[j, hi_sl] * rC[j, hi_sl]
                    )
                if j >= _D:
                    pbuf[j - _D, lo] = pending.pop(j - _D)
            pltpu.sync_copy(pbuf, p_out.at[pl.ds(base + coff, _CHUNK)])
            return carry

        lax.fori_loop(0, n_chunks, chunk_body, 0)

    return sc_fn


@functools.lru_cache(maxsize=None)
def _tc_linear(n, k, feat):
    blk = 1024

    def mm(p_ref, w_ref, b_ref, o_ref):
        o_ref[...] = (
            jnp.dot(p_ref[...], w_ref[...], preferred_element_type=jnp.float32)
            + b_ref[...]
        )

    return pl.pallas_call(
        mm,
        grid=(n // blk,),
        in_specs=[
            pl.BlockSpec((blk, k), lambda i: (i, 0)),
            pl.BlockSpec((k, feat), lambda i: (0, 0)),
            pl.BlockSpec((1, feat), lambda i: (0, 0)),
        ],
        out_specs=pl.BlockSpec((blk, feat), lambda i: (i, 0)),
        out_shape=jax.ShapeDtypeStruct((n, feat), jnp.float32),
    )


def kernel(x_idx, y_idx, t_idx, A, B, C, W, b):
    rank, n = x_idx.shape
    table_rows = A.shape[1]
    feat = W.shape[0]
    total = rank * n

    xf = x_idx.T.reshape(total)
    yf = y_idx.T.reshape(total)
    tf = t_idx.T.reshape(total)
    At = A.T
    Bt = B.T
    Ct = C.T

    p = _sc_gather_prod(total, rank, table_rows)(xf, yf, tf, At, Bt, Ct)

    w2 = jnp.broadcast_to(W.T[:, None, :], (rank, _L, feat)).reshape(rank * _L, feat)
    return _tc_linear(n, rank * _L, feat)(p.reshape(n, rank * _L), w2, b.reshape(1, feat))


# speedup vs baseline: 1.0398x; 1.0398x over previous
import functools

import jax
import jax.numpy as jnp
from jax import lax
from jax.experimental import pallas as pl
from jax.experimental.pallas import tpu as pltpu
from jax.experimental.pallas import tpu_sc as plsc

_L = 16
_CHUNK = 128
_D = 8


@functools.lru_cache(maxsize=None)
def _sc_gather_prod(total, rank, table_rows):
    info = plsc.get_sparse_core_info()
    num_workers = info.num_cores * info.num_subcores
    per_w = total // num_workers
    assert per_w % _CHUNK == 0
    n_chunks = per_w // _CHUNK

    mesh = plsc.VectorSubcoreMesh(core_axis_name="c", subcore_axis_name="s")

    @functools.partial(
        pl.kernel,
        mesh=mesh,
        compiler_params=pltpu.CompilerParams(use_tc_tiling_on_sc=False),
        out_type=jax.ShapeDtypeStruct((total, _L), jnp.float32),
        scratch_types=[
            pltpu.VMEM((per_w,), jnp.float32),
            pltpu.VMEM((per_w,), jnp.float32),
            pltpu.VMEM((per_w,), jnp.float32),
            pltpu.VMEM((_CHUNK,), jnp.int32),
            pltpu.VMEM((_CHUNK,), jnp.int32),
            pltpu.VMEM((_CHUNK,), jnp.int32),
            pltpu.VMEM((_CHUNK, rank), jnp.float32),
            pltpu.VMEM((_CHUNK, rank), jnp.float32),
            pltpu.VMEM((_CHUNK, rank), jnp.float32),
            pltpu.VMEM((_CHUNK, _L), jnp.float32),
            pltpu.SemaphoreType.DMA,
        ],
    )
    def sc_fn(xf, yf, tf, At, Bt, Ct, p_out,
              fx, fy, ft, ix, iy, it, rA, rB, rC, pbuf, sem):
        wid = lax.axis_index("s") * info.num_cores + lax.axis_index("c")
        base = wid * per_w
        pltpu.sync_copy(xf.at[pl.ds(base, per_w)], fx)
        pltpu.sync_copy(yf.at[pl.ds(base, per_w)], fy)
        pltpu.sync_copy(tf.at[pl.ds(base, per_w)], ft)

        xscale = jnp.float32(table_rows - 1)
        yscale = jnp.float32(table_rows)
        hi = table_rows - 1
        lo = pl.ds(0, _L)
        hi_sl = pl.ds(_L, _L)

        def chunk_body(c, carry):
            coff = c * _CHUNK
            for gi in range(_CHUNK // _L):
                src = pl.ds(coff + gi * _L, _L)
                dst = pl.ds(gi * _L, _L)
                ix[dst] = jnp.clip((fx[src] * xscale).astype(jnp.int32), 0, hi)
                iy[dst] = jnp.clip((fy[src] * yscale - 1.0).astype(jnp.int32), 0, hi)
                it[dst] = jnp.clip((ft[src] * yscale - 1.0).astype(jnp.int32), 0, hi)
            ca = pltpu.async_copy(At.at[ix], rA, sem)
            cb = pltpu.async_copy(Bt.at[iy], rB, sem)
            cc = pltpu.async_copy(Ct.at[it], rC, sem)
            ca.wait()
            cb.wait()
            cc.wait()
            for j in range(_CHUNK):
                p = (rA[j, lo] * rB[j, lo] * rC[j, lo]
                     + rA[j, hi_sl] * rB[j, hi_sl] * rC[j, hi_sl])
                pbuf[j, lo] = p
            pltpu.sync_copy(pbuf, p_out.at[pl.ds(base + coff, _CHUNK)])
            return carry

        lax.fori_loop(0, n_chunks, chunk_body, 0)

    return sc_fn


@functools.lru_cache(maxsize=None)
def _tc_linear(n, k, feat):
    blk = 1024

    def mm(p_ref, w_ref, b_ref, o_ref):
        o_ref[...] = (
            jnp.dot(p_ref[...], w_ref[...], preferred_element_type=jnp.float32)
            + b_ref[...]
        )

    return pl.pallas_call(
        mm,
        grid=(n // blk,),
        in_specs=[
            pl.BlockSpec((blk, k), lambda i: (i, 0)),
            pl.BlockSpec((k, feat), lambda i: (0, 0)),
            pl.BlockSpec((1, feat), lambda i: (0, 0)),
        ],
        out_specs=pl.BlockSpec((blk, feat), lambda i: (i, 0)),
        out_shape=jax.ShapeDtypeStruct((n, feat), jnp.float32),
    )


def kernel(x_idx, y_idx, t_idx, A, B, C, W, b):
    rank, n = x_idx.shape
    table_rows = A.shape[1]
    feat = W.shape[0]
    total = rank * n

    xf = x_idx.T.reshape(total)
    yf = y_idx.T.reshape(total)
    tf = t_idx.T.reshape(total)
    At = A.T
    Bt = B.T
    Ct = C.T

    p = _sc_gather_prod(total, rank, table_rows)(xf, yf, tf, At, Bt, Ct)

    w2 = jnp.broadcast_to(W.T[:, None, :], (rank, _L, feat)).reshape(rank * _L, feat)
    return _tc_linear(n, rank * _L, feat)(p.reshape(n, rank * _L), w2, b.reshape(1, feat))


# speedup vs baseline: 1.7785x; 1.7105x over previous
import functools

import jax
import jax.numpy as jnp
from jax import lax
from jax.experimental import pallas as pl
from jax.experimental.pallas import tpu as pltpu
from jax.experimental.pallas import tpu_sc as plsc

_L = 16
_CHUNK = 128


@functools.lru_cache(maxsize=None)
def _sc_gather_prod(total, rank, table_rows):
    info = plsc.get_sparse_core_info()
    num_workers = info.num_cores * info.num_subcores
    per_w = total // num_workers
    assert per_w % (2 * _CHUNK) == 0
    n_chunks = per_w // _CHUNK

    mesh = plsc.VectorSubcoreMesh(core_axis_name="c", subcore_axis_name="s")

    buf = lambda shape, dt: pltpu.VMEM(shape, dt)

    @functools.partial(
        pl.kernel,
        mesh=mesh,
        compiler_params=pltpu.CompilerParams(use_tc_tiling_on_sc=False),
        out_type=jax.ShapeDtypeStruct((total, _L), jnp.float32),
        scratch_types=[
            buf((per_w,), jnp.float32),
            buf((per_w,), jnp.float32),
            buf((per_w,), jnp.float32),
            [buf((_CHUNK,), jnp.int32) for _ in range(2)],
            [buf((_CHUNK,), jnp.int32) for _ in range(2)],
            [buf((_CHUNK,), jnp.int32) for _ in range(2)],
            [buf((_CHUNK, rank), jnp.float32) for _ in range(2)],
            [buf((_CHUNK, rank), jnp.float32) for _ in range(2)],
            [buf((_CHUNK, rank), jnp.float32) for _ in range(2)],
            [buf((_CHUNK, _L), jnp.float32) for _ in range(2)],
            [pltpu.SemaphoreType.DMA for _ in range(2)],
            [pltpu.SemaphoreType.DMA for _ in range(2)],
        ],
    )
    def sc_fn(xf, yf, tf, At, Bt, Ct, p_out,
              fx, fy, ft, ix, iy, it, rA, rB, rC, pbuf, semg, semw):
        wid = lax.axis_index("s") * info.num_cores + lax.axis_index("c")
        base = wid * per_w
        pltpu.sync_copy(xf.at[pl.ds(base, per_w)], fx)
        pltpu.sync_copy(yf.at[pl.ds(base, per_w)], fy)
        pltpu.sync_copy(tf.at[pl.ds(base, per_w)], ft)

        xscale = jnp.float32(table_rows - 1)
        yscale = jnp.float32(table_rows)
        hi = table_rows - 1

        def quantize(c, b):
            coff = c * _CHUNK

            def qbody(gi, carry):
                src = pl.ds(coff + gi * _L, _L)
                dst = pl.ds(gi * _L, _L)
                ix[b][dst] = jnp.clip((fx[src] * xscale).astype(jnp.int32), 0, hi)
                iy[b][dst] = jnp.clip((fy[src] * yscale - 1.0).astype(jnp.int32), 0, hi)
                it[b][dst] = jnp.clip((ft[src] * yscale - 1.0).astype(jnp.int32), 0, hi)
                return carry

            lax.fori_loop(0, _CHUNK // _L, qbody, 0, unroll=2)

        def fire(b):
            pltpu.async_copy(At.at[ix[b]], rA[b], semg[b])
            pltpu.async_copy(Bt.at[iy[b]], rB[b], semg[b])
            pltpu.async_copy(Ct.at[it[b]], rC[b], semg[b])

        def waitg(b):
            pltpu.make_async_copy(At.at[ix[b]], rA[b], semg[b]).wait()
            pltpu.make_async_copy(Bt.at[iy[b]], rB[b], semg[b]).wait()
            pltpu.make_async_copy(Ct.at[it[b]], rC[b], semg[b]).wait()

        lo = pl.ds(0, _L)
        hi_sl = pl.ds(_L, _L)

        def compute(b):
            def cbody(j, carry):
                p = (rA[b][j, lo] * rB[b][j, lo] * rC[b][j, lo]
                     + rA[b][j, hi_sl] * rB[b][j, hi_sl] * rC[b][j, hi_sl])
                pbuf[b][j, lo] = p
                return carry

            lax.fori_loop(0, _CHUNK, cbody, 0, unroll=8)

        def fire_out(c, b):
            pltpu.async_copy(pbuf[b], p_out.at[pl.ds(base + c * _CHUNK, _CHUNK)], semw[b])

        def wait_out(c, b):
            pltpu.make_async_copy(
                pbuf[b], p_out.at[pl.ds(base + c * _CHUNK, _CHUNK)], semw[b]
            ).wait()

        quantize(0, 0)
        fire(0)
        quantize(1, 1)
        fire(1)
        waitg(0)
        compute(0)
        fire_out(0, 0)

        def pair_body(c2, carry):
            c0 = c2 * 2
            quantize(c0, 0)
            fire(0)
            waitg(1)

            @pl.when(c2 >= 2)
            def _():
                wait_out(c0 - 3, 1)

            compute(1)
            fire_out(c0 - 1, 1)
            quantize(c0 + 1, 1)
            fire(1)
            waitg(0)
            wait_out(c0 - 2, 0)
            compute(0)
            fire_out(c0, 0)
            return carry

        lax.fori_loop(1, n_chunks // 2, pair_body, 0)

        waitg(1)
        wait_out(n_chunks - 3, 1)
        compute(1)
        fire_out(n_chunks - 1, 1)
        wait_out(n_chunks - 2, 0)
        wait_out(n_chunks - 1, 1)

    return sc_fn


@functools.lru_cache(maxsize=None)
def _tc_linear(n, k, feat):
    blk = 1024

    def mm(p_ref, w_ref, b_ref, o_ref):
        o_ref[...] = (
            jnp.dot(p_ref[...], w_ref[...], preferred_element_type=jnp.float32)
            + b_ref[...]
        )

    return pl.pallas_call(
        mm,
        grid=(n // blk,),
        in_specs=[
            pl.BlockSpec((blk, k), lambda i: (i, 0)),
            pl.BlockSpec((k, feat), lambda i: (0, 0)),
            pl.BlockSpec((1, feat), lambda i: (0, 0)),
        ],
        out_specs=pl.BlockSpec((blk, feat), lambda i: (i, 0)),
        out_shape=jax.ShapeDtypeStruct((n, feat), jnp.float32),
    )


def kernel(x_idx, y_idx, t_idx, A, B, C, W, b):
    rank, n = x_idx.shape
    table_rows = A.shape[1]
    feat = W.shape[0]
    total = rank * n

    xf = x_idx.T.reshape(total)
    yf = y_idx.T.reshape(total)
    tf = t_idx.T.reshape(total)
    At = A.T
    Bt = B.T
    Ct = C.T

    p = _sc_gather_prod(total, rank, table_rows)(xf, yf, tf, At, Bt, Ct)

    w2 = jnp.broadcast_to(W.T[:, None, :], (rank, _L, feat)).reshape(rank * _L, feat)
    return _tc_linear(n, rank * _L, feat)(p.reshape(n, rank * _L), w2, b.reshape(1, feat))


# speedup vs baseline: 2.1179x; 1.1908x over previous
import functools

import jax
import jax.numpy as jnp
from jax import lax
from jax.experimental import pallas as pl
from jax.experimental.pallas import tpu as pltpu
from jax.experimental.pallas import tpu_sc as plsc

_L = 16
_CHUNK = 128


@functools.lru_cache(maxsize=None)
def _sc_gather_prod(total, rank, table_rows):
    info = plsc.get_sparse_core_info()
    num_workers = info.num_cores * info.num_subcores
    per_w = total // num_workers
    assert per_w % (2 * _CHUNK) == 0
    n_chunks = per_w // _CHUNK

    mesh = plsc.VectorSubcoreMesh(core_axis_name="c", subcore_axis_name="s")

    buf = lambda shape, dt: pltpu.VMEM(shape, dt)

    @functools.partial(
        pl.kernel,
        mesh=mesh,
        compiler_params=pltpu.CompilerParams(use_tc_tiling_on_sc=False),
        out_type=jax.ShapeDtypeStruct((total, _L), jnp.float32),
        scratch_types=[
            buf((per_w,), jnp.float32),
            buf((per_w,), jnp.float32),
            buf((per_w,), jnp.float32),
            [buf((_CHUNK,), jnp.int32) for _ in range(2)],
            [buf((_CHUNK,), jnp.int32) for _ in range(2)],
            [buf((_CHUNK,), jnp.int32) for _ in range(2)],
            [buf((_CHUNK, rank), jnp.float32) for _ in range(2)],
            [buf((_CHUNK, rank), jnp.float32) for _ in range(2)],
            [buf((_CHUNK, rank), jnp.float32) for _ in range(2)],
            [buf((_CHUNK, _L), jnp.float32) for _ in range(2)],
            [pltpu.SemaphoreType.DMA for _ in range(2)],
            [pltpu.SemaphoreType.DMA for _ in range(2)],
        ],
    )
    def sc_fn(xf, yf, tf, At, Bt, Ct, p_out,
              fx, fy, ft, ix, iy, it, rA, rB, rC, pbuf, semg, semw):
        wid = lax.axis_index("s") * info.num_cores + lax.axis_index("c")
        base = wid * per_w
        pltpu.sync_copy(xf.at[pl.ds(base, per_w)], fx)
        pltpu.sync_copy(yf.at[pl.ds(base, per_w)], fy)
        pltpu.sync_copy(tf.at[pl.ds(base, per_w)], ft)

        xscale = jnp.float32(table_rows - 1)
        yscale = jnp.float32(table_rows)
        hi = table_rows - 1

        def quantize(c, b):
            coff = c * _CHUNK

            @plsc.parallel_loop(0, _CHUNK // _L, unroll=4)
            def _(gi):
                src = pl.ds(coff + gi * _L, _L)
                dst = pl.ds(gi * _L, _L)
                ix[b][dst] = jnp.clip((fx[src] * xscale).astype(jnp.int32), 0, hi)
                iy[b][dst] = jnp.clip((fy[src] * yscale - 1.0).astype(jnp.int32), 0, hi)
                it[b][dst] = jnp.clip((ft[src] * yscale - 1.0).astype(jnp.int32), 0, hi)

        def fire(b):
            pltpu.async_copy(At.at[ix[b]], rA[b], semg[b])
            pltpu.async_copy(Bt.at[iy[b]], rB[b], semg[b])
            pltpu.async_copy(Ct.at[it[b]], rC[b], semg[b])

        def waitg(b):
            pltpu.make_async_copy(At.at[ix[b]], rA[b], semg[b]).wait()
            pltpu.make_async_copy(Bt.at[iy[b]], rB[b], semg[b]).wait()
            pltpu.make_async_copy(Ct.at[it[b]], rC[b], semg[b]).wait()

        lo = pl.ds(0, _L)
        hi_sl = pl.ds(_L, _L)

        def compute(b):
            @plsc.parallel_loop(0, _CHUNK, unroll=8)
            def _(j):
                p = (rA[b][j, lo] * rB[b][j, lo] * rC[b][j, lo]
                     + rA[b][j, hi_sl] * rB[b][j, hi_sl] * rC[b][j, hi_sl])
                pbuf[b][j, lo] = p

        def fire_out(c, b):
            pltpu.async_copy(pbuf[b], p_out.at[pl.ds(base + c * _CHUNK, _CHUNK)], semw[b])

        def wait_out(c, b):
            pltpu.make_async_copy(
                pbuf[b], p_out.at[pl.ds(base + c * _CHUNK, _CHUNK)], semw[b]
            ).wait()

        quantize(0, 0)
        fire(0)
        quantize(1, 1)
        fire(1)
        waitg(0)
        compute(0)
        fire_out(0, 0)

        def pair_body(c2, carry):
            c0 = c2 * 2
            quantize(c0, 0)
            fire(0)
            waitg(1)

            @pl.when(c2 >= 2)
            def _():
                wait_out(c0 - 3, 1)

            compute(1)
            fire_out(c0 - 1, 1)
            quantize(c0 + 1, 1)
            fire(1)
            waitg(0)
            wait_out(c0 - 2, 0)
            compute(0)
            fire_out(c0, 0)
            return carry

        lax.fori_loop(1, n_chunks // 2, pair_body, 0)

        waitg(1)
        wait_out(n_chunks - 3, 1)
        compute(1)
        fire_out(n_chunks - 1, 1)
        wait_out(n_chunks - 2, 0)
        wait_out(n_chunks - 1, 1)

    return sc_fn


@functools.lru_cache(maxsize=None)
def _tc_linear(n, k, feat):
    blk = 1024

    def mm(p_ref, w_ref, b_ref, o_ref):
        o_ref[...] = (
            jnp.dot(p_ref[...], w_ref[...], preferred_element_type=jnp.float32)
            + b_ref[...]
        )

    return pl.pallas_call(
        mm,
        grid=(n // blk,),
        in_specs=[
            pl.BlockSpec((blk, k), lambda i: (i, 0)),
            pl.BlockSpec((k, feat), lambda i: (0, 0)),
            pl.BlockSpec((1, feat), lambda i: (0, 0)),
        ],
        out_specs=pl.BlockSpec((blk, feat), lambda i: (i, 0)),
        out_shape=jax.ShapeDtypeStruct((n, feat), jnp.float32),
    )


def kernel(x_idx, y_idx, t_idx, A, B, C, W, b):
    rank, n = x_idx.shape
    table_rows = A.shape[1]
    feat = W.shape[0]
    total = rank * n

    xf = x_idx.T.reshape(total)
    yf = y_idx.T.reshape(total)
    tf = t_idx.T.reshape(total)
    At = A.T
    Bt = B.T
    Ct = C.T

    p = _sc_gather_prod(total, rank, table_rows)(xf, yf, tf, At, Bt, Ct)

    w2 = jnp.broadcast_to(W.T[:, None, :], (rank, _L, feat)).reshape(rank * _L, feat)
    return _tc_linear(n, rank * _L, feat)(p.reshape(n, rank * _L), w2, b.reshape(1, feat))
